# baseline (speedup 1.0000x reference)
import functools

import jax
import jax.numpy as jnp
import numpy as np
from jax import lax
from jax.experimental import pallas as pl
from jax.experimental.pallas import tpu as pltpu
from jax.experimental.pallas import tpu_sc as plsc

_K = 2
_AUX_COEF = 0.01
_BM = 256
_NW = 32


def _router_body(x_ref, wrt_ref, i1_ref, i2_ref, w1_ref, w2_ref, aux_ref):
    xx = x_ref[...]
    wrt = wrt_ref[...]
    l = jnp.dot(xx, wrt, preferred_element_type=jnp.float32)
    e = l.shape[1]
    m = jnp.max(l, axis=-1, keepdims=True)
    ex = jnp.exp(l - m)
    p = ex / jnp.sum(ex, axis=-1, keepdims=True)
    col = lax.broadcasted_iota(jnp.int32, p.shape, 1)
    p1 = jnp.max(p, axis=-1, keepdims=True)
    i1 = jnp.min(jnp.where(p == p1, col, e), axis=-1, keepdims=True)
    pm = jnp.where(col == i1, -1.0, p)
    p2 = jnp.max(pm, axis=-1, keepdims=True)
    i2 = jnp.min(jnp.where(pm == p2, col, e), axis=-1, keepdims=True)
    sw = p1 + p2
    i1_ref[...] = i1
    i2_ref[...] = i2
    w1_ref[...] = p1 / sw
    w2_ref[...] = p2 / sw
    cnt = (jnp.sum((col == i1).astype(jnp.float32), axis=0, keepdims=True)
           + jnp.sum((col == i2).astype(jnp.float32), axis=0, keepdims=True))
    tpe = cnt / p.shape[0]
    rpp = jnp.mean(p, axis=0, keepdims=True)
    aux_ref[...] = (jnp.sum(tpe * rpp) * e * _AUX_COEF).reshape(1, 1)


def _router(xf, wr):
    s, d = xf.shape
    return pl.pallas_call(
        _router_body,
        out_shape=(
            jax.ShapeDtypeStruct((s, 1), jnp.int32),
            jax.ShapeDtypeStruct((s, 1), jnp.int32),
            jax.ShapeDtypeStruct((s, 1), jnp.float32),
            jax.ShapeDtypeStruct((s, 1), jnp.float32),
            jax.ShapeDtypeStruct((1, 1), jnp.float32),
        ),
    )(xf, wr.T)


def _ffn_body(bid_ref, eid_ref, off_ref,
              xs_ref, wg_ref, wu_ref, wd_ref, wbc_ref, ys_ref,
              wgb_ref, wub_ref, wdb_ref):
    i = pl.program_id(0)
    ee = eid_ref[i]
    b = bid_ref[i]

    @pl.when((i == 0) | (eid_ref[jnp.maximum(i - 1, 0)] != ee))
    def _cast_weights():
        wgb_ref[...] = wg_ref[0].astype(jnp.bfloat16)
        wub_ref[...] = wu_ref[0].astype(jnp.bfloat16)
        wdb_ref[...] = wd_ref[0].astype(jnp.bfloat16)

    x = xs_ref[...].astype(jnp.bfloat16)
    nt = (((1,), (1,)), ((), ()))
    g = lax.dot_general(x, wgb_ref[...], nt, preferred_element_type=jnp.float32)
    u = lax.dot_general(x, wub_ref[...], nt, preferred_element_type=jnp.float32)
    h = (g * jax.nn.sigmoid(g) * u).astype(jnp.bfloat16)
    y = lax.dot_general(h, wdb_ref[...], nt, preferred_element_type=jnp.float32)
    y = y * wbc_ref[...][:, :1]
    r = b * _BM + lax.broadcasted_iota(jnp.int32, (_BM, 1), 0)
    mask = (r >= off_ref[ee]) & (r < off_ref[ee + 1])
    ys_ref[...] = jnp.where(mask, y, ys_ref[...])


def _ffn(bids, eids, offsets, xs, wg, wu, wd, wbc):
    n, d = xs.shape
    e, ff, _ = wg.shape
    g = bids.shape[0]
    spec = pltpu.PrefetchScalarGridSpec(
        num_scalar_prefetch=3,
        grid=(g,),
        in_specs=[
            pl.BlockSpec((_BM, d), lambda i, bi, ei, of: (bi[i], 0)),
            pl.BlockSpec((1, ff, d), lambda i, bi, ei, of: (ei[i], 0, 0)),
            pl.BlockSpec((1, ff, d), lambda i, bi, ei, of: (ei[i], 0, 0)),
            pl.BlockSpec((1, d, ff), lambda i, bi, ei, of: (ei[i], 0, 0)),
            pl.BlockSpec((_BM, 128), lambda i, bi, ei, of: (bi[i], 0)),
        ],
        out_specs=pl.BlockSpec((_BM, d), lambda i, bi, ei, of: (bi[i], 0)),
        scratch_shapes=[
            pltpu.VMEM((ff, d), jnp.bfloat16),
            pltpu.VMEM((ff, d), jnp.bfloat16),
            pltpu.VMEM((d, ff), jnp.bfloat16),
        ],
    )
    return pl.pallas_call(
        _ffn_body,
        grid_spec=spec,
        out_shape=jax.ShapeDtypeStruct((n, d), jnp.float32),
        compiler_params=pltpu.CompilerParams(
            dimension_semantics=("arbitrary",)),
    )(bids, eids, offsets, xs, wg, wu, wd, wbc)


def _sc_mesh():
    return plsc.VectorSubcoreMesh(core_axis_name="c", subcore_axis_name="s")


def _sc_wid():
    return lax.axis_index("s") * 2 + lax.axis_index("c")


def _gather_rows(xf, src):
    s, d = xf.shape
    n = src.shape[0]
    rpw = n // _NW

    @functools.partial(
        pl.kernel,
        out_type=jax.ShapeDtypeStruct((n, d), jnp.float32),
        mesh=_sc_mesh(),
        scratch_types=[
            pltpu.VMEM((rpw,), jnp.int32),
            pltpu.VMEM((rpw, d), jnp.float32),
            pltpu.SemaphoreType.DMA,
        ],
    )
    def gather_k(xf_hbm, idx_hbm, out_hbm, idx_v, rows_v, sem):
        base = _sc_wid() * rpw
        pltpu.sync_copy(idx_hbm.at[pl.ds(base, rpw)], idx_v)
        pltpu.async_copy(xf_hbm.at[idx_v], rows_v, sem).wait()
        pltpu.sync_copy(rows_v, out_hbm.at[pl.ds(base, rpw)])

    return gather_k(xf, src)


def _combine_rows(ys, pos_a, pos_b):
    n, d = ys.shape
    s = pos_a.shape[0]
    tpw = s // _NW
    nch = d // 16

    @functools.partial(
        pl.kernel,
        out_type=jax.ShapeDtypeStruct((s, d), jnp.float32),
        mesh=_sc_mesh(),
        scratch_types=[
            pltpu.VMEM((tpw,), jnp.int32),
            pltpu.VMEM((tpw,), jnp.int32),
            pltpu.VMEM((tpw, d), jnp.float32),
            pltpu.VMEM((tpw, d), jnp.float32),
            pltpu.SemaphoreType.DMA,
            pltpu.SemaphoreType.DMA,
        ],
    )
    def combine_k(ys_hbm, pa_hbm, pb_hbm, out_hbm,
                  ia_v, ib_v, ra_v, rb_v, sa, sb):
        base = _sc_wid() * tpw
        pltpu.sync_copy(pa_hbm.at[pl.ds(base, tpw)], ia_v)
        pltpu.sync_copy(pb_hbm.at[pl.ds(base, tpw)], ib_v)
        ca = pltpu.async_copy(ys_hbm.at[ia_v], ra_v, sa)
        cb = pltpu.async_copy(ys_hbm.at[ib_v], rb_v, sb)
        ca.wait()
        cb.wait()

        def tok_body(t, carry):
            for c in range(nch):
                sl = pl.ds(c * 16, 16)
                ra_v[t, sl] = ra_v[t, sl] + rb_v[t, sl]
            return carry

        lax.fori_loop(0, tpw, tok_body, 0)
        pltpu.sync_copy(ra_v, out_hbm.at[pl.ds(base, tpw)])

    return combine_k(ys, pos_a, pos_b)


def kernel(x, Wg, Wu, Wd, Wr):
    bq, sq, d = x.shape
    e, ff, _ = Wg.shape
    s = bq * sq
    n = s * _K
    xf = x.reshape(s, d)

    i1, i2, w1, w2, aux11 = _router(xf, Wr)

    e_flat = jnp.concatenate([i1, i2], axis=1).reshape(n)
    w_flat = jnp.concatenate([w1, w2], axis=1).reshape(n)
    perm = jnp.argsort(e_flat).astype(jnp.int32)
    pos = jnp.zeros((n,), jnp.int32).at[perm].set(
        jnp.arange(n, dtype=jnp.int32))
    src = (perm // _K).astype(jnp.int32)
    w_sorted = w_flat[perm]
    counts = jnp.zeros((e,), jnp.int32).at[e_flat].add(1)
    offsets = jnp.concatenate(
        [jnp.zeros((1,), jnp.int32), jnp.cumsum(counts).astype(jnp.int32)])

    nb = n // _BM
    gsz = nb + e - 1
    b_all = jnp.asarray(np.repeat(np.arange(nb), e), jnp.int32)
    e_all = jnp.asarray(np.tile(np.arange(e), nb), jnp.int32)
    valid = ((offsets[e_all] < (b_all + 1) * _BM)
             & (offsets[e_all + 1] > b_all * _BM))
    order = jnp.argsort(jnp.logical_not(valid).astype(jnp.int32), stable=True)
    bv = b_all[order]
    ev = e_all[order]
    nvalid = jnp.sum(valid.astype(jnp.int32))
    lastb = bv[nvalid - 1]
    laste = ev[nvalid - 1]
    ar = jnp.arange(gsz)
    bids = jnp.where(ar < nvalid, bv[:gsz], lastb).astype(jnp.int32)
    eids = jnp.where(ar < nvalid, ev[:gsz], laste).astype(jnp.int32)

    wbc = jnp.broadcast_to(w_sorted[:, None], (n, 128))

    xs = _gather_rows(xf, src)
    ys = _ffn(bids, eids, offsets, xs, Wg, Wu, Wd, wbc)
    pos2 = pos.reshape(s, _K)
    out = _combine_rows(ys, pos2[:, 0], pos2[:, 1])

    return out.reshape(bq, sq, d), aux11.reshape(())


# speedup vs baseline: 1.2186x; 1.2186x over previous
import functools

import jax
import jax.numpy as jnp
from jax import lax
from jax.experimental import pallas as pl
from jax.experimental.pallas import tpu as pltpu
from jax.experimental.pallas import tpu_sc as plsc

_K = 2
_AUX_COEF = 0.01
_BM = 256
_NW = 32


def _router_body(x_ref, wrt_ref, ids_ref, ws_ref, cnt_ref, aux_ref):
    xx = x_ref[...]
    wrt = wrt_ref[...]
    l = jnp.dot(xx, wrt, preferred_element_type=jnp.float32)
    e = l.shape[1]
    m = jnp.max(l, axis=-1, keepdims=True)
    ex = jnp.exp(l - m)
    p = ex / jnp.sum(ex, axis=-1, keepdims=True)
    col = lax.broadcasted_iota(jnp.int32, p.shape, 1)
    p1 = jnp.max(p, axis=-1, keepdims=True)
    i1 = jnp.min(jnp.where(p == p1, col, e), axis=-1, keepdims=True)
    pm = jnp.where(col == i1, -1.0, p)
    p2 = jnp.max(pm, axis=-1, keepdims=True)
    i2 = jnp.min(jnp.where(pm == p2, col, e), axis=-1, keepdims=True)
    sw = p1 + p2
    ids_ref[:, 0:1] = i1
    ids_ref[:, 1:2] = i2
    ws_ref[:, 0:1] = p1 / sw
    ws_ref[:, 1:2] = p2 / sw
    cnt = (jnp.sum((col == i1).astype(jnp.float32), axis=0, keepdims=True)
           + jnp.sum((col == i2).astype(jnp.float32), axis=0, keepdims=True))
    cnt_ref[...] = cnt.astype(jnp.int32)
    tpe = cnt / p.shape[0]
    rpp = jnp.mean(p, axis=0, keepdims=True)
    aux_ref[...] = (jnp.sum(tpe * rpp) * e * _AUX_COEF).reshape(1, 1)


def _router(xf, wr):
    s, d = xf.shape
    e = wr.shape[0]
    return pl.pallas_call(
        _router_body,
        out_shape=(
            jax.ShapeDtypeStruct((s, _K), jnp.int32),
            jax.ShapeDtypeStruct((s, _K), jnp.float32),
            jax.ShapeDtypeStruct((1, e), jnp.int32),
            jax.ShapeDtypeStruct((1, 1), jnp.float32),
        ),
    )(xf, wr.T)


def _ffn_body(bid_ref, eid_ref, off_ref,
              xs_ref, wg_ref, wu_ref, wd_ref, wbc_ref, ys_ref):
    i = pl.program_id(0)
    ee = eid_ref[i]
    b = bid_ref[i]
    x = xs_ref[...]
    nt = (((1,), (1,)), ((), ()))
    g = lax.dot_general(x, wg_ref[0], nt, preferred_element_type=jnp.float32)
    u = lax.dot_general(x, wu_ref[0], nt, preferred_element_type=jnp.float32)
    h = g * jax.nn.sigmoid(g) * u
    y = lax.dot_general(h, wd_ref[0], nt, preferred_element_type=jnp.float32)
    y = y * wbc_ref[...]
    r = b * _BM + lax.broadcasted_iota(jnp.int32, (_BM, 1), 0)
    mask = (r >= off_ref[ee]) & (r < off_ref[ee + 1])
    ys_ref[...] = jnp.where(mask, y, ys_ref[...])


def _ffn(bids, eids, offsets, xs, wg, wu, wd, wbc):
    n, d = xs.shape
    e, ff, _ = wg.shape
    g = bids.shape[0]
    spec = pltpu.PrefetchScalarGridSpec(
        num_scalar_prefetch=3,
        grid=(g,),
        in_specs=[
            pl.BlockSpec((_BM, d), lambda i, bi, ei, of: (bi[i], 0)),
            pl.BlockSpec((1, ff, d), lambda i, bi, ei, of: (ei[i], 0, 0)),
            pl.BlockSpec((1, ff, d), lambda i, bi, ei, of: (ei[i], 0, 0)),
            pl.BlockSpec((1, d, ff), lambda i, bi, ei, of: (ei[i], 0, 0)),
            pl.BlockSpec((_BM, 1), lambda i, bi, ei, of: (bi[i], 0)),
        ],
        out_specs=pl.BlockSpec((_BM, d), lambda i, bi, ei, of: (bi[i], 0)),
    )
    return pl.pallas_call(
        _ffn_body,
        grid_spec=spec,
        out_shape=jax.ShapeDtypeStruct((n, d), jnp.float32),
        compiler_params=pltpu.CompilerParams(
            dimension_semantics=("arbitrary",)),
    )(bids, eids, offsets, xs, wg, wu, wd, wbc)


def _sc_mesh():
    return plsc.VectorSubcoreMesh(core_axis_name="c", subcore_axis_name="s")


def _sc_wid():
    return lax.axis_index("s") * 2 + lax.axis_index("c")


def _gather_rows(xf, src):
    s, d = xf.shape
    n = src.shape[0]
    rpw = n // _NW

    @functools.partial(
        pl.kernel,
        out_type=jax.ShapeDtypeStruct((n, d), jnp.float32),
        mesh=_sc_mesh(),
        scratch_types=[
            pltpu.VMEM((rpw,), jnp.int32),
            pltpu.VMEM((rpw, d), jnp.float32),
            pltpu.SemaphoreType.DMA,
        ],
    )
    def gather_k(xf_hbm, idx_hbm, out_hbm, idx_v, rows_v, sem):
        base = _sc_wid() * rpw
        pltpu.sync_copy(idx_hbm.at[pl.ds(base, rpw)], idx_v)
        pltpu.async_copy(xf_hbm.at[idx_v], rows_v, sem).wait()
        pltpu.sync_copy(rows_v, out_hbm.at[pl.ds(base, rpw)])

    return gather_k(xf, src)


def _combine_rows(ys, pos_a, pos_b):
    n, d = ys.shape
    s = pos_a.shape[0]
    tpw = s // _NW
    nch = d // 16

    @functools.partial(
        pl.kernel,
        out_type=jax.ShapeDtypeStruct((s, d), jnp.float32),
        mesh=_sc_mesh(),
        scratch_types=[
            pltpu.VMEM((tpw,), jnp.int32),
            pltpu.VMEM((tpw,), jnp.int32),
            pltpu.VMEM((tpw, d), jnp.float32),
            pltpu.VMEM((tpw, d), jnp.float32),
            pltpu.SemaphoreType.DMA,
            pltpu.SemaphoreType.DMA,
        ],
    )
    def combine_k(ys_hbm, pa_hbm, pb_hbm, out_hbm,
                  ia_v, ib_v, ra_v, rb_v, sa, sb):
        base = _sc_wid() * tpw
        pltpu.sync_copy(pa_hbm.at[pl.ds(base, tpw)], ia_v)
        pltpu.sync_copy(pb_hbm.at[pl.ds(base, tpw)], ib_v)
        ca = pltpu.async_copy(ys_hbm.at[ia_v], ra_v, sa)
        cb = pltpu.async_copy(ys_hbm.at[ib_v], rb_v, sb)
        ca.wait()
        cb.wait()

        def tok_body(t, carry):
            for c in range(nch):
                sl = pl.ds(c * 16, 16)
                ra_v[t, sl] = ra_v[t, sl] + rb_v[t, sl]
            return carry

        lax.fori_loop(0, tpw, tok_body, 0)
        pltpu.sync_copy(ra_v, out_hbm.at[pl.ds(base, tpw)])

    return combine_k(ys, pos_a, pos_b)


def kernel(x, Wg, Wu, Wd, Wr):
    bq, sq, d = x.shape
    e, ff, _ = Wg.shape
    s = bq * sq
    n = s * _K
    xf = x.reshape(s, d)

    ids, ws, counts, aux11 = _router(xf, Wr)

    e_flat = ids.reshape(n)
    perm = jnp.argsort(e_flat).astype(jnp.int32)
    src = (perm // _K).astype(jnp.int32)
    pos = jnp.argsort(perm).astype(jnp.int32)
    w_sorted = ws.reshape(n)[perm].reshape(n, 1)
    offsets = jnp.concatenate(
        [jnp.zeros((1,), jnp.int32), jnp.cumsum(counts[0]).astype(jnp.int32)])

    nb = n // _BM
    cuts = jnp.sort(jnp.concatenate(
        [jnp.arange(nb, dtype=jnp.int32) * _BM, offsets[1:e]]))
    cuts = jnp.minimum(cuts, n - 1)
    bids = (cuts // _BM).astype(jnp.int32)
    eids = jnp.clip(jnp.searchsorted(offsets, cuts, side="right") - 1,
                    0, e - 1).astype(jnp.int32)

    xs = _gather_rows(xf, src)
    ys = _ffn(bids, eids, offsets, xs, Wg, Wu, Wd, w_sorted)
    pos2 = pos.reshape(s, _K)
    out = _combine_rows(ys, pos2[:, 0], pos2[:, 1])

    return out.reshape(bq, sq, d), aux11.reshape(())
